# baseline (speedup 1.0000x reference)
import functools

import jax
import jax.numpy as jnp
from jax.experimental import pallas as pl
from jax.experimental.pallas import tpu as pltpu

_K = 8
_ALPHA = 0.0001
_BT = 8192


def _router_body(x_ref, w_ref, b_ref, gate_ref, idx_ref, loss_ref, p_acc, f_acc):
    i = pl.program_id(0)
    n = pl.num_programs(0)
    bt = x_ref.shape[0]
    e = w_ref.shape[0]

    @pl.when(i == 0)
    def _init():
        p_acc[...] = jnp.zeros_like(p_acc)
        f_acc[...] = jnp.zeros_like(f_acc)

    logits_t = jax.lax.dot_general(
        w_ref[...], x_ref[...],
        (((1,), (1,)), ((), ())),
        preferred_element_type=jnp.float32,
    )
    a = jax.nn.sigmoid(logits_t)
    s = a + b_ref[...]

    inv_rowsum = 1.0 / (jnp.sum(a, axis=0, keepdims=True) + 1e-9)
    p_acc[...] += jnp.sum(a * inv_rowsum, axis=1, keepdims=True)

    iota_e = jax.lax.broadcasted_iota(jnp.int32, (e, bt), 0)
    neg = jnp.float32(-3.0e38)

    av_rows = []
    ix_rows = []
    for _ in range(_K):
        m = jnp.max(s, axis=0, keepdims=True)
        first = jnp.min(jnp.where(s == m, iota_e, e), axis=0, keepdims=True)
        s = jnp.where(iota_e == first, neg, s)
        ix_rows.append(first)
        av_rows.append(m)
    sel_total = (s <= jnp.float32(-1e38)).astype(jnp.float32)
    f_acc[...] += jnp.sum(sel_total, axis=1, keepdims=True)

    gates = jnp.concatenate(av_rows, axis=0)
    gsum = jnp.sum(gates, axis=0, keepdims=True) + 1e-9
    gate_ref[...] = gates / gsum
    idx_ref[...] = jnp.concatenate(ix_rows, axis=0)

    @pl.when(i == n - 1)
    def _finish():
        t = jnp.float32(n * bt)
        scale = _ALPHA * e / (_K * t * t)
        loss_ref[...] = (scale * jnp.sum(f_acc[...] * p_acc[...])).reshape(1, 1)


@functools.partial(jax.jit, static_argnames=("interpret",))
def kernel(x, W, expert_bias, interpret=False):
    t, d = x.shape
    e = W.shape[0]
    grid = (t // _BT,)
    gate_t, idx_t, loss = pl.pallas_call(
        _router_body,
        grid=grid,
        in_specs=[
            pl.BlockSpec((_BT, d), lambda i: (i, 0)),
            pl.BlockSpec((e, d), lambda i: (0, 0)),
            pl.BlockSpec((e, 1), lambda i: (0, 0)),
        ],
        out_specs=[
            pl.BlockSpec((_K, _BT), lambda i: (0, i)),
            pl.BlockSpec((_K, _BT), lambda i: (0, i)),
            pl.BlockSpec((1, 1), lambda i: (0, 0)),
        ],
        out_shape=[
            jax.ShapeDtypeStruct((_K, t), jnp.float32),
            jax.ShapeDtypeStruct((_K, t), jnp.int32),
            jax.ShapeDtypeStruct((1, 1), jnp.float32),
        ],
        scratch_shapes=[
            pltpu.VMEM((e, 1), jnp.float32),
            pltpu.VMEM((e, 1), jnp.float32),
        ],
        compiler_params=pltpu.CompilerParams(
            dimension_semantics=("arbitrary",),
        ),
        interpret=interpret,
    )(x, W, expert_bias.reshape(e, 1))
    return gate_t.T, idx_t.T, loss[0, 0]


# speedup vs baseline: 1.1589x; 1.1589x over previous
import functools

import jax
import jax.numpy as jnp
from jax.experimental import pallas as pl
from jax.experimental.pallas import tpu as pltpu

_K = 8
_ALPHA = 0.0001
_BT = 4096


def _router_body(x_ref, w_ref, b_ref, gate_ref, idx_ref, loss_ref, p_acc, f_acc):
    i = pl.program_id(0)
    n = pl.num_programs(0)
    bt = x_ref.shape[0]
    e = w_ref.shape[0]

    @pl.when(i == 0)
    def _init():
        p_acc[...] = jnp.zeros_like(p_acc)
        f_acc[...] = jnp.zeros_like(f_acc)

    logits_t = jax.lax.dot_general(
        w_ref[...], x_ref[...],
        (((1,), (1,)), ((), ())),
        preferred_element_type=jnp.float32,
    )
    a = jax.nn.sigmoid(logits_t)
    s = a + b_ref[...]

    inv_rowsum = 1.0 / (jnp.sum(a, axis=0, keepdims=True) + 1e-9)
    p_acc[...] += jnp.sum(a * inv_rowsum, axis=1, keepdims=True)

    iota_bf = jax.lax.broadcasted_iota(jnp.int32, (1, e), 1).astype(jnp.bfloat16)
    neg = jnp.float32(-3.0e38)

    av_rows = []
    eq_rows = []
    for _ in range(_K):
        m = jnp.max(s, axis=0, keepdims=True)
        eq = s == m
        s = jnp.where(eq, neg, s)
        eq_rows.append(eq.astype(jnp.bfloat16))
        av_rows.append(m)
    sel_total = (s <= jnp.float32(-1e38)).astype(jnp.float32)
    f_acc[...] += jnp.sum(sel_total, axis=1, keepdims=True)

    eq_all = jnp.concatenate(eq_rows, axis=1)
    ix_all = jax.lax.dot_general(
        iota_bf, eq_all, (((1,), (0,)), ((), ())),
        preferred_element_type=jnp.float32,
    )
    ix_rows = [
        ix_all[:, k * bt:(k + 1) * bt].astype(jnp.int32) for k in range(_K)
    ]

    gates = jnp.concatenate(av_rows, axis=0)
    gsum = jnp.sum(gates, axis=0, keepdims=True) + 1e-9
    gate_ref[...] = gates / gsum
    idx_ref[...] = jnp.concatenate(ix_rows, axis=0)

    @pl.when(i == n - 1)
    def _finish():
        t = jnp.float32(n * bt)
        scale = _ALPHA * e / (_K * t * t)
        loss_ref[...] = (scale * jnp.sum(f_acc[...] * p_acc[...])).reshape(1, 1)


@functools.partial(jax.jit, static_argnames=("interpret",))
def kernel(x, W, expert_bias, interpret=False):
    t, d = x.shape
    e = W.shape[0]
    grid = (t // _BT,)
    gate_t, idx_t, loss = pl.pallas_call(
        _router_body,
        grid=grid,
        in_specs=[
            pl.BlockSpec((_BT, d), lambda i: (i, 0)),
            pl.BlockSpec((e, d), lambda i: (0, 0)),
            pl.BlockSpec((e, 1), lambda i: (0, 0)),
        ],
        out_specs=[
            pl.BlockSpec((_K, _BT), lambda i: (0, i)),
            pl.BlockSpec((_K, _BT), lambda i: (0, i)),
            pl.BlockSpec((1, 1), lambda i: (0, 0)),
        ],
        out_shape=[
            jax.ShapeDtypeStruct((_K, t), jnp.float32),
            jax.ShapeDtypeStruct((_K, t), jnp.int32),
            jax.ShapeDtypeStruct((1, 1), jnp.float32),
        ],
        scratch_shapes=[
            pltpu.VMEM((e, 1), jnp.float32),
            pltpu.VMEM((e, 1), jnp.float32),
        ],
        compiler_params=pltpu.CompilerParams(
            dimension_semantics=("arbitrary",),
        ),
        interpret=interpret,
    )(x, W, expert_bias.reshape(e, 1))
    return gate_t.T, idx_t.T, loss[0, 0]
